# baseline (speedup 1.0000x reference)
import jax
import jax.numpy as jnp
from jax import lax
from jax.experimental import pallas as pl
from jax.experimental.pallas import tpu as pltpu
from jax.experimental.pallas import tpu_sc as plsc

BATCH = 16384
EMB = 32
NUM_CORES = 2
NUM_SUBCORES = 16
NUM_WORKERS = NUM_CORES * NUM_SUBCORES
BPW = BATCH // NUM_WORKERS
CHUNK = 128
NCHUNKS = BPW // CHUNK
LANES = 16


def _mf_body(user_hbm, item_hbm, utab_hbm, itab_hbm, out_hbm,
             uidx, iidx, urows16, irows16, urows, irows, outv, usem, isem):
    wid = lax.axis_index("s") * NUM_CORES + lax.axis_index("c")

    pltpu.sync_copy(user_hbm.at[wid], uidx)
    pltpu.sync_copy(item_hbm.at[wid], iidx)

    for c in range(NCHUNKS):
        ucp = pltpu.async_copy(
            utab_hbm.at[uidx.at[c]], urows16.at[pl.ds(c * CHUNK, CHUNK)], usem)
        icp = pltpu.async_copy(
            itab_hbm.at[iidx.at[c]], irows16.at[pl.ds(c * CHUNK, CHUNK)], isem)
        ucp.wait()
        icp.wait()

    @pl.loop(0, BPW)
    def _(i):
        ua, ub = plsc.unpack(urows16[i, :], format=plsc.PackFormat.INTERLEAVED)
        urows[i, pl.ds(0, LANES)] = ua
        urows[i, pl.ds(LANES, LANES)] = ub
        ia, ib = plsc.unpack(irows16[i, :], format=plsc.PackFormat.INTERLEAVED)
        irows[i, pl.ds(0, LANES)] = ia
        irows[i, pl.ds(LANES, LANES)] = ib

    @pl.loop(0, BPW, step=LANES)
    def _(i0):
        rows = lax.iota(jnp.int32, LANES) + i0
        cols0 = jnp.zeros((LANES,), jnp.int32)
        acc = (plsc.load_gather(urows, [rows, cols0]) *
               plsc.load_gather(irows, [rows, cols0]))
        for j in range(1, EMB):
            cols = jnp.full((LANES,), j, jnp.int32)
            acc = acc + (plsc.load_gather(urows, [rows, cols]) *
                         plsc.load_gather(irows, [rows, cols]))
        outv[pl.ds(i0, LANES)] = acc

    pltpu.sync_copy(outv, out_hbm.at[pl.ds(wid * BPW, BPW)])


@jax.jit
def _mf(user3, item3, utab16, itab16):
    mesh = plsc.VectorSubcoreMesh(core_axis_name="c", subcore_axis_name="s")
    run = pl.kernel(
        _mf_body,
        out_type=jax.ShapeDtypeStruct((BATCH,), jnp.float32),
        mesh=mesh,
        compiler_params=pltpu.CompilerParams(
            needs_layout_passes=False, use_tc_tiling_on_sc=False),
        scratch_types=[
            pltpu.VMEM((NCHUNKS, CHUNK), jnp.int32),
            pltpu.VMEM((NCHUNKS, CHUNK), jnp.int32),
            pltpu.VMEM((BPW, EMB), jnp.bfloat16),
            pltpu.VMEM((BPW, EMB), jnp.bfloat16),
            pltpu.VMEM((BPW, EMB), jnp.float32),
            pltpu.VMEM((BPW, EMB), jnp.float32),
            pltpu.VMEM((BPW,), jnp.float32),
            pltpu.SemaphoreType.DMA,
            pltpu.SemaphoreType.DMA,
        ],
    )
    return run(user3, item3, utab16, itab16)


def kernel(user, item, user_table, item_table):
    user3 = user.astype(jnp.int32).reshape(NUM_WORKERS, NCHUNKS, CHUNK)
    item3 = item.astype(jnp.int32).reshape(NUM_WORKERS, NCHUNKS, CHUNK)
    return _mf(user3, item3,
               user_table.astype(jnp.bfloat16), item_table.astype(jnp.bfloat16))


# speedup vs baseline: 1.1519x; 1.1519x over previous
import jax
import jax.numpy as jnp
from jax import lax
from jax.experimental import pallas as pl
from jax.experimental.pallas import tpu as pltpu
from jax.experimental.pallas import tpu_sc as plsc

BATCH = 16384
EMB = 32
NUM_CORES = 1
NUM_SUBCORES = 16
NUM_WORKERS = NUM_CORES * NUM_SUBCORES
BPW = BATCH // NUM_WORKERS
CHUNK = 128
NCHUNKS = BPW // CHUNK
LANES = 16


def _mf_body(user_hbm, item_hbm, utab_hbm, itab_hbm, out_hbm,
             uidx, iidx, urows, irows, outv, usem, isem):
    wid = lax.axis_index("s") * NUM_CORES + lax.axis_index("c")

    pltpu.sync_copy(user_hbm.at[wid], uidx)
    pltpu.sync_copy(item_hbm.at[wid], iidx)

    for c in range(NCHUNKS):
        ucp = pltpu.async_copy(
            utab_hbm.at[uidx.at[c]], urows.at[pl.ds(c * CHUNK, CHUNK)], usem)
        icp = pltpu.async_copy(
            itab_hbm.at[iidx.at[c]], irows.at[pl.ds(c * CHUNK, CHUNK)], isem)
        ucp.wait()
        icp.wait()

    @pl.loop(0, BPW, step=LANES)
    def _(i0):
        rows = lax.iota(jnp.int32, LANES) + i0
        cols0 = jnp.zeros((LANES,), jnp.int32)
        acc = (plsc.load_gather(urows, [rows, cols0]) *
               plsc.load_gather(irows, [rows, cols0]))
        for j in range(1, EMB):
            cols = jnp.full((LANES,), j, jnp.int32)
            acc = acc + (plsc.load_gather(urows, [rows, cols]) *
                         plsc.load_gather(irows, [rows, cols]))
        outv[pl.ds(i0, LANES)] = acc

    pltpu.sync_copy(outv, out_hbm.at[pl.ds(wid * BPW, BPW)])


@jax.jit
def _mf(user3, item3, user_table, item_table):
    mesh = plsc.VectorSubcoreMesh(core_axis_name="c", subcore_axis_name="s", num_cores=NUM_CORES)
    run = pl.kernel(
        _mf_body,
        out_type=jax.ShapeDtypeStruct((BATCH,), jnp.float32),
        mesh=mesh,
        compiler_params=pltpu.CompilerParams(
            needs_layout_passes=False, use_tc_tiling_on_sc=False),
        scratch_types=[
            pltpu.VMEM((NCHUNKS, CHUNK), jnp.int32),
            pltpu.VMEM((NCHUNKS, CHUNK), jnp.int32),
            pltpu.VMEM((BPW, EMB), jnp.float32),
            pltpu.VMEM((BPW, EMB), jnp.float32),
            pltpu.VMEM((BPW,), jnp.float32),
            pltpu.SemaphoreType.DMA,
            pltpu.SemaphoreType.DMA,
        ],
    )
    return run(user3, item3, user_table, item_table)


def kernel(user, item, user_table, item_table):
    user3 = user.astype(jnp.int32).reshape(NUM_WORKERS, NCHUNKS, CHUNK)
    item3 = item.astype(jnp.int32).reshape(NUM_WORKERS, NCHUNKS, CHUNK)
    return _mf(user3, item3, user_table, item_table)


# speedup vs baseline: 1.1765x; 1.0214x over previous
import jax
import jax.numpy as jnp
from jax import lax
from jax.experimental import pallas as pl
from jax.experimental.pallas import tpu as pltpu
from jax.experimental.pallas import tpu_sc as plsc

BATCH = 16384
EMB = 32
NUM_CORES = 2
NUM_SUBCORES = 16
NUM_WORKERS = NUM_CORES * NUM_SUBCORES
BPW = BATCH // NUM_WORKERS
CHUNK = 128
NCHUNKS = BPW // CHUNK
LANES = 16


def _mf_body(user_hbm, item_hbm, utab_hbm, itab_hbm, out_hbm,
             uidx, iidx, urows, irows, outv, usem, isem):
    wid = lax.axis_index("s") * NUM_CORES + lax.axis_index("c")

    pltpu.sync_copy(user_hbm.at[wid], uidx)
    pltpu.sync_copy(item_hbm.at[wid], iidx)

    for c in range(NCHUNKS):
        ucp = pltpu.async_copy(
            utab_hbm.at[uidx.at[c]], urows.at[pl.ds(c * CHUNK, CHUNK)], usem)
        icp = pltpu.async_copy(
            itab_hbm.at[iidx.at[c]], irows.at[pl.ds(c * CHUNK, CHUNK)], isem)
        ucp.wait()
        icp.wait()

    @pl.loop(0, BPW, step=LANES)
    def _(i0):
        rows = lax.iota(jnp.int32, LANES) + i0
        cols0 = jnp.zeros((LANES,), jnp.int32)
        acc = (plsc.load_gather(urows, [rows, cols0]) *
               plsc.load_gather(irows, [rows, cols0]))
        for j in range(1, EMB):
            cols = jnp.full((LANES,), j, jnp.int32)
            acc = acc + (plsc.load_gather(urows, [rows, cols]) *
                         plsc.load_gather(irows, [rows, cols]))
        outv[pl.ds(i0, LANES)] = acc

    pltpu.sync_copy(outv, out_hbm.at[pl.ds(wid * BPW, BPW)])


@jax.jit
def _mf(user3, item3, user_table, item_table):
    mesh = plsc.VectorSubcoreMesh(core_axis_name="c", subcore_axis_name="s")
    run = pl.kernel(
        _mf_body,
        out_type=jax.ShapeDtypeStruct((BATCH,), jnp.float32),
        mesh=mesh,
        compiler_params=pltpu.CompilerParams(
            needs_layout_passes=False, use_tc_tiling_on_sc=False),
        scratch_types=[
            pltpu.VMEM((NCHUNKS, CHUNK), jnp.int32),
            pltpu.VMEM((NCHUNKS, CHUNK), jnp.int32),
            pltpu.VMEM((BPW, EMB), jnp.float32),
            pltpu.VMEM((BPW, EMB), jnp.float32),
            pltpu.VMEM((BPW,), jnp.float32),
            pltpu.SemaphoreType.DMA,
            pltpu.SemaphoreType.DMA,
        ],
    )
    return run(user3, item3, user_table, item_table)


def kernel(user, item, user_table, item_table):
    user3 = user.astype(jnp.int32).reshape(NUM_WORKERS, NCHUNKS, CHUNK)
    item3 = item.astype(jnp.int32).reshape(NUM_WORKERS, NCHUNKS, CHUNK)
    return _mf(user3, item3, user_table, item_table)
